# Initial kernel scaffold; baseline (speedup 1.0000x reference)
import jax
import jax.numpy as jnp
from jax.experimental import pallas as pl


def kernel(atom_fea, nbr_fea, nbr_fea_idx, crystal_atom_idx, W_emb, b_emb, conv_W, conv_b, bn1_gamma, bn1_beta, bn2_gamma, bn2_beta, W_fc, b_fc, W_out1, b_out1, W_out2, b_out2):
    raise NotImplementedError("write your pallas kernel here")


# speedup vs baseline: 1.5695x; 1.5695x over previous
import functools

import jax
import jax.numpy as jnp
from jax import lax
from jax.experimental import pallas as pl
from jax.experimental.pallas import tpu as pltpu
from jax.experimental.pallas import tpu_sc as plsc

_N = 10000
_M = 32
_AFL = 64
_NBRF = 16
_NCONV = 3
_EPS = 1e-5
_NE = _N * _M

_NW = 32
_CH = 128
_CPW = 80
_EPW = _CH * _CPW
_EPAD = _NW * _EPW

_NBLK = 25
_APB = _N // _NBLK
_EPB = _APB * _M


def _sc_gather(table, idx2d):
    mesh = plsc.VectorSubcoreMesh(core_axis_name="c", subcore_axis_name="s")

    @functools.partial(
        pl.kernel,
        mesh=mesh,
        compiler_params=pltpu.CompilerParams(use_tc_tiling_on_sc=False),
        out_type=jax.ShapeDtypeStruct((_EPAD, _AFL), jnp.float32),
        scratch_types=[
            pltpu.VMEM((_CPW, _CH), jnp.int32),
            pltpu.VMEM((_CH, _AFL), jnp.float32),
            pltpu.SemaphoreType.DMA,
        ],
    )
    def k(table_hbm, idx_hbm, out_hbm, idx_v, rows_v, sem):
        wid = lax.axis_index("s") * 2 + lax.axis_index("c")
        pltpu.sync_copy(idx_hbm.at[pl.ds(wid * _CPW, _CPW)], idx_v)

        def body(ch, carry):
            pltpu.async_copy(table_hbm.at[idx_v.at[ch]], rows_v, sem).wait()
            pltpu.sync_copy(rows_v, out_hbm.at[pl.ds(wid * _EPW + ch * _CH, _CH)])
            return carry

        lax.fori_loop(0, _CPW, body, 0)

    return k(table, idx2d)


def _k_embed(atom_fea, W, b):
    def body(a_ref, w_ref, b_ref, o_ref):
        o_ref[...] = (
            jnp.dot(a_ref[...], w_ref[...], preferred_element_type=jnp.float32)
            + b_ref[...]
        )

    return pl.pallas_call(
        body,
        out_shape=jax.ShapeDtypeStruct((_N, _AFL), jnp.float32),
    )(atom_fea, W, b.reshape(1, _AFL))


def _edge_gated(x_blk, g_blk, e_blk, w1, w2, w3, b):
    P = jnp.dot(x_blk, w1, preferred_element_type=jnp.float32)
    g = jnp.dot(g_blk, w2, preferred_element_type=jnp.float32)
    g = g + jnp.dot(e_blk, w3, preferred_element_type=jnp.float32)
    g = g.reshape(_APB, _M, 2 * _AFL) + P[:, None, :] + b.reshape(1, 1, 2 * _AFL)
    return g.reshape(_EPB, 2 * _AFL)


def _k_stats(x, gathered, e2d, w1, w2, w3, b):
    def body(x_ref, g_ref, e_ref, w1_ref, w2_ref, w3_ref, b_ref, s1_ref, s2_ref):
        i = pl.program_id(0)
        g = _edge_gated(x_ref[...], g_ref[...], e_ref[...],
                        w1_ref[...], w2_ref[...], w3_ref[...], b_ref[...])
        s1_ref[pl.ds(i, 1), :] = jnp.sum(g, axis=0, keepdims=True)
        s2_ref[pl.ds(i, 1), :] = jnp.sum(g * g, axis=0, keepdims=True)

    return pl.pallas_call(
        body,
        grid=(_NBLK,),
        in_specs=[
            pl.BlockSpec((_APB, _AFL), lambda i: (i, 0)),
            pl.BlockSpec((_EPB, _AFL), lambda i: (i, 0)),
            pl.BlockSpec((_EPB, _NBRF), lambda i: (i, 0)),
            pl.BlockSpec((_AFL, 2 * _AFL), lambda i: (0, 0)),
            pl.BlockSpec((_AFL, 2 * _AFL), lambda i: (0, 0)),
            pl.BlockSpec((_NBRF, 2 * _AFL), lambda i: (0, 0)),
            pl.BlockSpec((1, 2 * _AFL), lambda i: (0, 0)),
        ],
        out_specs=[
            pl.BlockSpec((_NBLK, 2 * _AFL), lambda i: (0, 0)),
            pl.BlockSpec((_NBLK, 2 * _AFL), lambda i: (0, 0)),
        ],
        out_shape=[
            jax.ShapeDtypeStruct((_NBLK, 2 * _AFL), jnp.float32),
            jax.ShapeDtypeStruct((_NBLK, 2 * _AFL), jnp.float32),
        ],
    )(x, gathered, e2d, w1, w2, w3, b.reshape(1, 2 * _AFL))


def _k_apply(x, gathered, e2d, w1, w2, w3, b, s1, s2, gamma, beta):
    def body(x_ref, g_ref, e_ref, w1_ref, w2_ref, w3_ref, b_ref,
             s1_ref, s2_ref, gm_ref, bt_ref, o_ref):
        cnt = jnp.float32(_NE)
        mean = jnp.sum(s1_ref[...], axis=0, keepdims=True) / cnt
        var = jnp.sum(s2_ref[...], axis=0, keepdims=True) / cnt - mean * mean
        scale = lax.rsqrt(var + _EPS) * gm_ref[...]
        shift = bt_ref[...] - mean * scale
        g = _edge_gated(x_ref[...], g_ref[...], e_ref[...],
                        w1_ref[...], w2_ref[...], w3_ref[...], b_ref[...])
        g = g * scale + shift
        filt = jax.nn.sigmoid(g[:, :_AFL])
        core = jax.nn.softplus(g[:, _AFL:])
        prod = (filt * core).reshape(_APB, _M, _AFL)
        o_ref[...] = jnp.sum(prod, axis=1)

    return pl.pallas_call(
        body,
        grid=(_NBLK,),
        in_specs=[
            pl.BlockSpec((_APB, _AFL), lambda i: (i, 0)),
            pl.BlockSpec((_EPB, _AFL), lambda i: (i, 0)),
            pl.BlockSpec((_EPB, _NBRF), lambda i: (i, 0)),
            pl.BlockSpec((_AFL, 2 * _AFL), lambda i: (0, 0)),
            pl.BlockSpec((_AFL, 2 * _AFL), lambda i: (0, 0)),
            pl.BlockSpec((_NBRF, 2 * _AFL), lambda i: (0, 0)),
            pl.BlockSpec((1, 2 * _AFL), lambda i: (0, 0)),
            pl.BlockSpec((_NBLK, 2 * _AFL), lambda i: (0, 0)),
            pl.BlockSpec((_NBLK, 2 * _AFL), lambda i: (0, 0)),
            pl.BlockSpec((1, 2 * _AFL), lambda i: (0, 0)),
            pl.BlockSpec((1, 2 * _AFL), lambda i: (0, 0)),
        ],
        out_specs=pl.BlockSpec((_APB, _AFL), lambda i: (i, 0)),
        out_shape=jax.ShapeDtypeStruct((_N, _AFL), jnp.float32),
    )(x, gathered, e2d, w1, w2, w3, b.reshape(1, 2 * _AFL),
      s1, s2, gamma.reshape(1, 2 * _AFL), beta.reshape(1, 2 * _AFL))


def _k_bn2res(ns, x_prev, gamma, beta):
    def body(ns_ref, x_ref, gm_ref, bt_ref, o_ref):
        ns_v = ns_ref[...]
        m = jnp.mean(ns_v, axis=0, keepdims=True)
        d = ns_v - m
        v = jnp.mean(d * d, axis=0, keepdims=True)
        nsb = d * lax.rsqrt(v + _EPS) * gm_ref[...] + bt_ref[...]
        x_v = x_ref[...]
        o_ref[...] = jax.nn.softplus(x_v + nsb) + x_v

    return pl.pallas_call(
        body,
        out_shape=jax.ShapeDtypeStruct((_N, _AFL), jnp.float32),
    )(ns, x_prev, gamma.reshape(1, _AFL), beta.reshape(1, _AFL))


def _k_head(x, W_fc, b_fc, W_out1, b_out1, W_out2, b_out2):
    n0 = 200
    apc = 50

    def body(x_ref, wfc_ref, bfc_ref, w1_ref, b1_ref, w2_ref, b2_ref,
             out_ref, crys_ref):
        pooled = jnp.mean(x_ref[...].reshape(n0, apc, _AFL), axis=1)
        crys = (
            jnp.dot(jax.nn.softplus(pooled), wfc_ref[...],
                    preferred_element_type=jnp.float32)
            + bfc_ref[...]
        )
        crys = jax.nn.softplus(crys)
        crys_ref[...] = crys
        h = jax.nn.softplus(
            jnp.dot(crys, w1_ref[...], preferred_element_type=jnp.float32)
            + b1_ref[...]
        )
        out_ref[...] = (
            jnp.dot(h, w2_ref[...], preferred_element_type=jnp.float32)
            + b2_ref[...]
        )

    return pl.pallas_call(
        body,
        out_shape=[
            jax.ShapeDtypeStruct((n0, 1), jnp.float32),
            jax.ShapeDtypeStruct((n0, 2 * _AFL), jnp.float32),
        ],
    )(x, W_fc, b_fc.reshape(1, 2 * _AFL), W_out1, b_out1.reshape(1, _AFL),
      W_out2, b_out2.reshape(1, 1))


def kernel(atom_fea, nbr_fea, nbr_fea_idx, crystal_atom_idx, W_emb, b_emb,
           conv_W, conv_b, bn1_gamma, bn1_beta, bn2_gamma, bn2_beta,
           W_fc, b_fc, W_out1, b_out1, W_out2, b_out2):
    idx_flat = nbr_fea_idx.reshape(-1).astype(jnp.int32)
    idx2d = jnp.pad(idx_flat, (0, _EPAD - _NE)).reshape(_NW * _CPW, _CH)
    e2d = nbr_fea.reshape(_NE, _NBRF)

    x = _k_embed(atom_fea, W_emb, b_emb)
    for i in range(_NCONV):
        w1 = conv_W[i, :_AFL]
        w2 = conv_W[i, _AFL:2 * _AFL]
        w3 = conv_W[i, 2 * _AFL:]
        gathered = _sc_gather(x, idx2d)
        s1, s2 = _k_stats(x, gathered, e2d, w1, w2, w3, conv_b[i])
        ns = _k_apply(x, gathered, e2d, w1, w2, w3, conv_b[i], s1, s2,
                      bn1_gamma[i], bn1_beta[i])
        x = _k_bn2res(ns, x, bn2_gamma[i], bn2_beta[i])
    return _k_head(x, W_fc, b_fc, W_out1, b_out1, W_out2, b_out2)


# speedup vs baseline: 1.6871x; 1.0749x over previous
import functools

import jax
import jax.numpy as jnp
from jax import lax
from jax.experimental import pallas as pl
from jax.experimental.pallas import tpu as pltpu
from jax.experimental.pallas import tpu_sc as plsc

_N = 10000
_M = 32
_AFL = 64
_NBRF = 16
_NCONV = 3
_EPS = 1e-5
_NE = _N * _M

_NW = 32
_CH = 128
_CPW = 80
_EPW = _CH * _CPW
_EPAD = _NW * _EPW

_NBLK = 25
_APB = _N // _NBLK
_EPB = _APB * _M


_KG = 5
_NBUF = 2 * _KG


def _sc_gather(table, idx2d):
    mesh = plsc.VectorSubcoreMesh(core_axis_name="c", subcore_axis_name="s")

    @functools.partial(
        pl.kernel,
        mesh=mesh,
        compiler_params=pltpu.CompilerParams(use_tc_tiling_on_sc=False),
        out_type=jax.ShapeDtypeStruct((_EPAD, _AFL), jnp.float32),
        scratch_types=[
            pltpu.VMEM((_CPW, _CH), jnp.int32),
            pltpu.VMEM((_NBUF, _CH, _AFL), jnp.float32),
            pltpu.SemaphoreType.DMA,
            pltpu.SemaphoreType.DMA,
            pltpu.SemaphoreType.DMA,
        ],
    )
    def k(table_hbm, idx_hbm, out_hbm, idx_v, rows_v, sem_g, sem_w0, sem_w1):
        wid = lax.axis_index("s") * 2 + lax.axis_index("c")
        pltpu.sync_copy(idx_hbm.at[pl.ds(wid * _CPW, _CPW)], idx_v)
        base = wid * _EPW

        def wcopy(ch, b, sem):
            return pltpu.make_async_copy(
                rows_v.at[b], out_hbm.at[pl.ds(base + ch * _CH, _CH)], sem)

        def run_group(grp, bufs, wsem, prev_grp, do_drain):
            @pl.when(do_drain)
            def _():
                for j in range(_KG):
                    wcopy(prev_grp * _KG + j, bufs[j], wsem).wait()
            for j in range(_KG):
                ch = grp * _KG + j
                pltpu.async_copy(
                    table_hbm.at[idx_v.at[ch]], rows_v.at[bufs[j]], sem_g)
            for j in range(_KG):
                ch = grp * _KG + j
                pltpu.make_async_copy(
                    table_hbm.at[idx_v.at[ch]], rows_v.at[bufs[j]], sem_g
                ).wait()
            for j in range(_KG):
                wcopy(grp * _KG + j, bufs[j], wsem).start()

        bank0 = list(range(_KG))
        bank1 = list(range(_KG, _NBUF))

        def body(p, carry):
            run_group(2 * p, bank0, sem_w0, 2 * p - 2, p >= 1)
            run_group(2 * p + 1, bank1, sem_w1, 2 * p - 1, p >= 1)
            return carry

        npair = _CPW // (2 * _KG)
        lax.fori_loop(0, npair, body, 0)
        last = npair * 2 - 2
        for j in range(_KG):
            wcopy(last * _KG + j, bank0[j], sem_w0).wait()
        for j in range(_KG):
            wcopy((last + 1) * _KG + j, bank1[j], sem_w1).wait()

    return k(table, idx2d)


def _k_embed(atom_fea, W, b):
    def body(a_ref, w_ref, b_ref, o_ref):
        o_ref[...] = (
            jnp.dot(a_ref[...], w_ref[...], preferred_element_type=jnp.float32)
            + b_ref[...]
        )

    return pl.pallas_call(
        body,
        out_shape=jax.ShapeDtypeStruct((_N, _AFL), jnp.float32),
    )(atom_fea, W, b.reshape(1, _AFL))


def _edge_gated(x_blk, g_blk, e_blk, w1, w2, w3, b):
    P = jnp.dot(x_blk, w1, preferred_element_type=jnp.float32)
    g = jnp.dot(g_blk, w2, preferred_element_type=jnp.float32)
    g = g + jnp.dot(e_blk, w3, preferred_element_type=jnp.float32)
    g = g.reshape(_APB, _M, 2 * _AFL) + P[:, None, :] + b.reshape(1, 1, 2 * _AFL)
    return g.reshape(_EPB, 2 * _AFL)


def _k_stats(x, gathered, e2d, w1, w2, w3, b):
    def body(x_ref, g_ref, e_ref, w1_ref, w2_ref, w3_ref, b_ref, s1_ref, s2_ref):
        i = pl.program_id(0)
        g = _edge_gated(x_ref[...], g_ref[...], e_ref[...],
                        w1_ref[...], w2_ref[...], w3_ref[...], b_ref[...])
        s1_ref[pl.ds(i, 1), :] = jnp.sum(g, axis=0, keepdims=True)
        s2_ref[pl.ds(i, 1), :] = jnp.sum(g * g, axis=0, keepdims=True)

    return pl.pallas_call(
        body,
        grid=(_NBLK,),
        in_specs=[
            pl.BlockSpec((_APB, _AFL), lambda i: (i, 0)),
            pl.BlockSpec((_EPB, _AFL), lambda i: (i, 0)),
            pl.BlockSpec((_EPB, _NBRF), lambda i: (i, 0)),
            pl.BlockSpec((_AFL, 2 * _AFL), lambda i: (0, 0)),
            pl.BlockSpec((_AFL, 2 * _AFL), lambda i: (0, 0)),
            pl.BlockSpec((_NBRF, 2 * _AFL), lambda i: (0, 0)),
            pl.BlockSpec((1, 2 * _AFL), lambda i: (0, 0)),
        ],
        out_specs=[
            pl.BlockSpec((_NBLK, 2 * _AFL), lambda i: (0, 0)),
            pl.BlockSpec((_NBLK, 2 * _AFL), lambda i: (0, 0)),
        ],
        out_shape=[
            jax.ShapeDtypeStruct((_NBLK, 2 * _AFL), jnp.float32),
            jax.ShapeDtypeStruct((_NBLK, 2 * _AFL), jnp.float32),
        ],
    )(x, gathered, e2d, w1, w2, w3, b.reshape(1, 2 * _AFL))


def _k_apply(x, gathered, e2d, w1, w2, w3, b, s1, s2, gamma, beta):
    def body(x_ref, g_ref, e_ref, w1_ref, w2_ref, w3_ref, b_ref,
             s1_ref, s2_ref, gm_ref, bt_ref, o_ref):
        cnt = jnp.float32(_NE)
        mean = jnp.sum(s1_ref[...], axis=0, keepdims=True) / cnt
        var = jnp.sum(s2_ref[...], axis=0, keepdims=True) / cnt - mean * mean
        scale = lax.rsqrt(var + _EPS) * gm_ref[...]
        shift = bt_ref[...] - mean * scale
        g = _edge_gated(x_ref[...], g_ref[...], e_ref[...],
                        w1_ref[...], w2_ref[...], w3_ref[...], b_ref[...])
        g = g * scale + shift
        filt = jax.nn.sigmoid(g[:, :_AFL])
        core = jax.nn.softplus(g[:, _AFL:])
        prod = (filt * core).reshape(_APB, _M, _AFL)
        o_ref[...] = jnp.sum(prod, axis=1)

    return pl.pallas_call(
        body,
        grid=(_NBLK,),
        in_specs=[
            pl.BlockSpec((_APB, _AFL), lambda i: (i, 0)),
            pl.BlockSpec((_EPB, _AFL), lambda i: (i, 0)),
            pl.BlockSpec((_EPB, _NBRF), lambda i: (i, 0)),
            pl.BlockSpec((_AFL, 2 * _AFL), lambda i: (0, 0)),
            pl.BlockSpec((_AFL, 2 * _AFL), lambda i: (0, 0)),
            pl.BlockSpec((_NBRF, 2 * _AFL), lambda i: (0, 0)),
            pl.BlockSpec((1, 2 * _AFL), lambda i: (0, 0)),
            pl.BlockSpec((_NBLK, 2 * _AFL), lambda i: (0, 0)),
            pl.BlockSpec((_NBLK, 2 * _AFL), lambda i: (0, 0)),
            pl.BlockSpec((1, 2 * _AFL), lambda i: (0, 0)),
            pl.BlockSpec((1, 2 * _AFL), lambda i: (0, 0)),
        ],
        out_specs=pl.BlockSpec((_APB, _AFL), lambda i: (i, 0)),
        out_shape=jax.ShapeDtypeStruct((_N, _AFL), jnp.float32),
    )(x, gathered, e2d, w1, w2, w3, b.reshape(1, 2 * _AFL),
      s1, s2, gamma.reshape(1, 2 * _AFL), beta.reshape(1, 2 * _AFL))


def _k_bn2res(ns, x_prev, gamma, beta):
    def body(ns_ref, x_ref, gm_ref, bt_ref, o_ref):
        ns_v = ns_ref[...]
        m = jnp.mean(ns_v, axis=0, keepdims=True)
        d = ns_v - m
        v = jnp.mean(d * d, axis=0, keepdims=True)
        nsb = d * lax.rsqrt(v + _EPS) * gm_ref[...] + bt_ref[...]
        x_v = x_ref[...]
        o_ref[...] = jax.nn.softplus(x_v + nsb) + x_v

    return pl.pallas_call(
        body,
        out_shape=jax.ShapeDtypeStruct((_N, _AFL), jnp.float32),
    )(ns, x_prev, gamma.reshape(1, _AFL), beta.reshape(1, _AFL))


def _k_head(x, W_fc, b_fc, W_out1, b_out1, W_out2, b_out2):
    n0 = 200
    apc = 50

    def body(x_ref, wfc_ref, bfc_ref, w1_ref, b1_ref, w2_ref, b2_ref,
             out_ref, crys_ref):
        pooled = jnp.mean(x_ref[...].reshape(n0, apc, _AFL), axis=1)
        crys = (
            jnp.dot(jax.nn.softplus(pooled), wfc_ref[...],
                    preferred_element_type=jnp.float32)
            + bfc_ref[...]
        )
        crys = jax.nn.softplus(crys)
        crys_ref[...] = crys
        h = jax.nn.softplus(
            jnp.dot(crys, w1_ref[...], preferred_element_type=jnp.float32)
            + b1_ref[...]
        )
        out_ref[...] = (
            jnp.dot(h, w2_ref[...], preferred_element_type=jnp.float32)
            + b2_ref[...]
        )

    return pl.pallas_call(
        body,
        out_shape=[
            jax.ShapeDtypeStruct((n0, 1), jnp.float32),
            jax.ShapeDtypeStruct((n0, 2 * _AFL), jnp.float32),
        ],
    )(x, W_fc, b_fc.reshape(1, 2 * _AFL), W_out1, b_out1.reshape(1, _AFL),
      W_out2, b_out2.reshape(1, 1))


def kernel(atom_fea, nbr_fea, nbr_fea_idx, crystal_atom_idx, W_emb, b_emb,
           conv_W, conv_b, bn1_gamma, bn1_beta, bn2_gamma, bn2_beta,
           W_fc, b_fc, W_out1, b_out1, W_out2, b_out2):
    idx_flat = nbr_fea_idx.reshape(-1).astype(jnp.int32)
    idx2d = jnp.pad(idx_flat, (0, _EPAD - _NE)).reshape(_NW * _CPW, _CH)
    e2d = nbr_fea.reshape(_NE, _NBRF)

    x = _k_embed(atom_fea, W_emb, b_emb)
    for i in range(_NCONV):
        w1 = conv_W[i, :_AFL]
        w2 = conv_W[i, _AFL:2 * _AFL]
        w3 = conv_W[i, 2 * _AFL:]
        gathered = _sc_gather(x, idx2d)
        s1, s2 = _k_stats(x, gathered, e2d, w1, w2, w3, conv_b[i])
        ns = _k_apply(x, gathered, e2d, w1, w2, w3, conv_b[i], s1, s2,
                      bn1_gamma[i], bn1_beta[i])
        x = _k_bn2res(ns, x, bn2_gamma[i], bn2_beta[i])
    return _k_head(x, W_fc, b_fc, W_out1, b_out1, W_out2, b_out2)


# speedup vs baseline: 1.7954x; 1.0642x over previous
import functools

import jax
import jax.numpy as jnp
from jax import lax
from jax.experimental import pallas as pl
from jax.experimental.pallas import tpu as pltpu
from jax.experimental.pallas import tpu_sc as plsc

_N = 10000
_M = 32
_AFL = 64
_NBRF = 16
_NCONV = 3
_EPS = 1e-5
_NE = _N * _M

_NW = 32
_CH = 128
_CPW = 80
_EPW = _CH * _CPW
_EPAD = _NW * _EPW

_NBLK = 25
_APB = _N // _NBLK
_EPB = _APB * _M


_KG = 8
_NBUF = 2 * _KG


def _sc_gather(table, idx2d):
    mesh = plsc.VectorSubcoreMesh(core_axis_name="c", subcore_axis_name="s")

    @functools.partial(
        pl.kernel,
        mesh=mesh,
        compiler_params=pltpu.CompilerParams(use_tc_tiling_on_sc=False),
        out_type=jax.ShapeDtypeStruct((_EPAD, _AFL), jnp.bfloat16),
        scratch_types=[
            pltpu.VMEM((_CPW, _CH), jnp.int32),
            pltpu.VMEM((_NBUF, _CH, _AFL), jnp.bfloat16),
            pltpu.SemaphoreType.DMA,
            pltpu.SemaphoreType.DMA,
            pltpu.SemaphoreType.DMA,
        ],
    )
    def k(table_hbm, idx_hbm, out_hbm, idx_v, rows_v, sem_g, sem_w0, sem_w1):
        wid = lax.axis_index("s") * 2 + lax.axis_index("c")
        pltpu.sync_copy(idx_hbm.at[pl.ds(wid * _CPW, _CPW)], idx_v)
        base = wid * _EPW

        def wcopy(ch, b, sem):
            return pltpu.make_async_copy(
                rows_v.at[b], out_hbm.at[pl.ds(base + ch * _CH, _CH)], sem)

        def run_group(grp, bufs, wsem, prev_grp, do_drain):
            @pl.when(do_drain)
            def _():
                for j in range(_KG):
                    wcopy(prev_grp * _KG + j, bufs[j], wsem).wait()
            for j in range(_KG):
                ch = grp * _KG + j
                pltpu.async_copy(
                    table_hbm.at[idx_v.at[ch]], rows_v.at[bufs[j]], sem_g)
            for j in range(_KG):
                ch = grp * _KG + j
                pltpu.make_async_copy(
                    table_hbm.at[idx_v.at[ch]], rows_v.at[bufs[j]], sem_g
                ).wait()
            for j in range(_KG):
                wcopy(grp * _KG + j, bufs[j], wsem).start()

        bank0 = list(range(_KG))
        bank1 = list(range(_KG, _NBUF))

        def body(p, carry):
            run_group(2 * p, bank0, sem_w0, 2 * p - 2, p >= 1)
            run_group(2 * p + 1, bank1, sem_w1, 2 * p - 1, p >= 1)
            return carry

        npair = _CPW // (2 * _KG)
        lax.fori_loop(0, npair, body, 0)
        last = npair * 2 - 2
        for j in range(_KG):
            wcopy(last * _KG + j, bank0[j], sem_w0).wait()
        for j in range(_KG):
            wcopy((last + 1) * _KG + j, bank1[j], sem_w1).wait()

    return k(table, idx2d)


def _k_embed(atom_fea, W, b):
    def body(a_ref, w_ref, b_ref, o_ref, ob_ref):
        x = (
            jnp.dot(a_ref[...], w_ref[...], preferred_element_type=jnp.float32)
            + b_ref[...]
        )
        o_ref[...] = x
        ob_ref[...] = x.astype(jnp.bfloat16)

    return pl.pallas_call(
        body,
        out_shape=[
            jax.ShapeDtypeStruct((_N, _AFL), jnp.float32),
            jax.ShapeDtypeStruct((_N, _AFL), jnp.bfloat16),
        ],
    )(atom_fea, W, b.reshape(1, _AFL))


def _edge_gated(x_blk, g_blk, e_blk, w1, w2, w3, b):
    P = jnp.dot(x_blk, w1, preferred_element_type=jnp.float32)
    g = jnp.dot(g_blk.astype(jnp.float32), w2,
                preferred_element_type=jnp.float32)
    g = g + jnp.dot(e_blk.astype(jnp.float32), w3,
                    preferred_element_type=jnp.float32)
    g = g.reshape(_APB, _M, 2 * _AFL) + P[:, None, :] + b.reshape(1, 1, 2 * _AFL)
    return g.reshape(_EPB, 2 * _AFL)


def _k_stats(x, gathered, e2d, w1, w2, w3, b):
    def body(x_ref, g_ref, e_ref, w1_ref, w2_ref, w3_ref, b_ref, s1_ref, s2_ref):
        i = pl.program_id(0)
        g = _edge_gated(x_ref[...], g_ref[...], e_ref[...],
                        w1_ref[...], w2_ref[...], w3_ref[...], b_ref[...])
        s1_ref[pl.ds(i, 1), :] = jnp.sum(g, axis=0, keepdims=True)
        s2_ref[pl.ds(i, 1), :] = jnp.sum(g * g, axis=0, keepdims=True)

    return pl.pallas_call(
        body,
        grid=(_NBLK,),
        in_specs=[
            pl.BlockSpec((_APB, _AFL), lambda i: (i, 0)),
            pl.BlockSpec((_EPB, _AFL), lambda i: (i, 0)),
            pl.BlockSpec((_EPB, _NBRF), lambda i: (i, 0)),
            pl.BlockSpec((_AFL, 2 * _AFL), lambda i: (0, 0)),
            pl.BlockSpec((_AFL, 2 * _AFL), lambda i: (0, 0)),
            pl.BlockSpec((_NBRF, 2 * _AFL), lambda i: (0, 0)),
            pl.BlockSpec((1, 2 * _AFL), lambda i: (0, 0)),
        ],
        out_specs=[
            pl.BlockSpec((_NBLK, 2 * _AFL), lambda i: (0, 0)),
            pl.BlockSpec((_NBLK, 2 * _AFL), lambda i: (0, 0)),
        ],
        out_shape=[
            jax.ShapeDtypeStruct((_NBLK, 2 * _AFL), jnp.float32),
            jax.ShapeDtypeStruct((_NBLK, 2 * _AFL), jnp.float32),
        ],
    )(x, gathered, e2d, w1, w2, w3, b.reshape(1, 2 * _AFL))


def _k_apply(x, gathered, e2d, w1, w2, w3, b, s1, s2, gamma, beta):
    def body(x_ref, g_ref, e_ref, w1_ref, w2_ref, w3_ref, b_ref,
             s1_ref, s2_ref, gm_ref, bt_ref, o_ref):
        cnt = jnp.float32(_NE)
        mean = jnp.sum(s1_ref[...], axis=0, keepdims=True) / cnt
        var = jnp.sum(s2_ref[...], axis=0, keepdims=True) / cnt - mean * mean
        scale = lax.rsqrt(var + _EPS) * gm_ref[...]
        shift = bt_ref[...] - mean * scale
        g = _edge_gated(x_ref[...], g_ref[...], e_ref[...],
                        w1_ref[...], w2_ref[...], w3_ref[...], b_ref[...])
        g = g * scale + shift
        filt = jax.nn.sigmoid(g[:, :_AFL])
        core = jax.nn.softplus(g[:, _AFL:])
        prod = (filt * core).reshape(_APB, _M, _AFL)
        o_ref[...] = jnp.sum(prod, axis=1)

    return pl.pallas_call(
        body,
        grid=(_NBLK,),
        in_specs=[
            pl.BlockSpec((_APB, _AFL), lambda i: (i, 0)),
            pl.BlockSpec((_EPB, _AFL), lambda i: (i, 0)),
            pl.BlockSpec((_EPB, _NBRF), lambda i: (i, 0)),
            pl.BlockSpec((_AFL, 2 * _AFL), lambda i: (0, 0)),
            pl.BlockSpec((_AFL, 2 * _AFL), lambda i: (0, 0)),
            pl.BlockSpec((_NBRF, 2 * _AFL), lambda i: (0, 0)),
            pl.BlockSpec((1, 2 * _AFL), lambda i: (0, 0)),
            pl.BlockSpec((_NBLK, 2 * _AFL), lambda i: (0, 0)),
            pl.BlockSpec((_NBLK, 2 * _AFL), lambda i: (0, 0)),
            pl.BlockSpec((1, 2 * _AFL), lambda i: (0, 0)),
            pl.BlockSpec((1, 2 * _AFL), lambda i: (0, 0)),
        ],
        out_specs=pl.BlockSpec((_APB, _AFL), lambda i: (i, 0)),
        out_shape=jax.ShapeDtypeStruct((_N, _AFL), jnp.float32),
    )(x, gathered, e2d, w1, w2, w3, b.reshape(1, 2 * _AFL),
      s1, s2, gamma.reshape(1, 2 * _AFL), beta.reshape(1, 2 * _AFL))


def _k_bn2res(ns, x_prev, gamma, beta):
    def body(ns_ref, x_ref, gm_ref, bt_ref, o_ref, ob_ref):
        ns_v = ns_ref[...]
        m = jnp.mean(ns_v, axis=0, keepdims=True)
        d = ns_v - m
        v = jnp.mean(d * d, axis=0, keepdims=True)
        nsb = d * lax.rsqrt(v + _EPS) * gm_ref[...] + bt_ref[...]
        x_v = x_ref[...]
        x_new = jax.nn.softplus(x_v + nsb) + x_v
        o_ref[...] = x_new
        ob_ref[...] = x_new.astype(jnp.bfloat16)

    return pl.pallas_call(
        body,
        out_shape=[
            jax.ShapeDtypeStruct((_N, _AFL), jnp.float32),
            jax.ShapeDtypeStruct((_N, _AFL), jnp.bfloat16),
        ],
    )(ns, x_prev, gamma.reshape(1, _AFL), beta.reshape(1, _AFL))


def _k_head(x, W_fc, b_fc, W_out1, b_out1, W_out2, b_out2):
    n0 = 200
    apc = 50

    def body(x_ref, wfc_ref, bfc_ref, w1_ref, b1_ref, w2_ref, b2_ref,
             out_ref, crys_ref):
        pooled = jnp.mean(x_ref[...].reshape(n0, apc, _AFL), axis=1)
        crys = (
            jnp.dot(jax.nn.softplus(pooled), wfc_ref[...],
                    preferred_element_type=jnp.float32)
            + bfc_ref[...]
        )
        crys = jax.nn.softplus(crys)
        crys_ref[...] = crys
        h = jax.nn.softplus(
            jnp.dot(crys, w1_ref[...], preferred_element_type=jnp.float32)
            + b1_ref[...]
        )
        out_ref[...] = (
            jnp.dot(h, w2_ref[...], preferred_element_type=jnp.float32)
            + b2_ref[...]
        )

    return pl.pallas_call(
        body,
        out_shape=[
            jax.ShapeDtypeStruct((n0, 1), jnp.float32),
            jax.ShapeDtypeStruct((n0, 2 * _AFL), jnp.float32),
        ],
    )(x, W_fc, b_fc.reshape(1, 2 * _AFL), W_out1, b_out1.reshape(1, _AFL),
      W_out2, b_out2.reshape(1, 1))


def kernel(atom_fea, nbr_fea, nbr_fea_idx, crystal_atom_idx, W_emb, b_emb,
           conv_W, conv_b, bn1_gamma, bn1_beta, bn2_gamma, bn2_beta,
           W_fc, b_fc, W_out1, b_out1, W_out2, b_out2):
    idx_flat = nbr_fea_idx.reshape(-1).astype(jnp.int32)
    idx2d = jnp.pad(idx_flat, (0, _EPAD - _NE)).reshape(_NW * _CPW, _CH)
    e2d = nbr_fea.reshape(_NE, _NBRF).astype(jnp.bfloat16)

    x, xb = _k_embed(atom_fea, W_emb, b_emb)
    for i in range(_NCONV):
        w1 = conv_W[i, :_AFL]
        w2 = conv_W[i, _AFL:2 * _AFL]
        w3 = conv_W[i, 2 * _AFL:]
        gathered = _sc_gather(xb, idx2d)
        s1, s2 = _k_stats(x, gathered, e2d, w1, w2, w3, conv_b[i])
        ns = _k_apply(x, gathered, e2d, w1, w2, w3, conv_b[i], s1, s2,
                      bn1_gamma[i], bn1_beta[i])
        x, xb = _k_bn2res(ns, x, bn2_gamma[i], bn2_beta[i])
    return _k_head(x, W_fc, b_fc, W_out1, b_out1, W_out2, b_out2)
